# Initial kernel scaffold; baseline (speedup 1.0000x reference)
import jax
import jax.numpy as jnp
from jax.experimental import pallas as pl


def kernel(x, edge_index, batch, W1, b1, g1, be1, W2, b2, g2, be2, W3, b3):
    raise NotImplementedError("write your pallas kernel here")


# speedup vs baseline: 26.0210x; 26.0210x over previous
import functools

import jax
import jax.numpy as jnp
from jax import lax
from jax.experimental import pallas as pl
from jax.experimental.pallas import tpu as pltpu
from jax.experimental.pallas import tpu_sc as plsc

_N = 10000
_NP = 10240
_E = 320000
_NC = 2
_NS = 16
_NW = _NC * _NS
_EPW = _E // _NW
_K = 125
_NBAT = _EPW // _K
_RPT = _NP // _NS


@functools.lru_cache(maxsize=None)
def _mesh():
    return plsc.VectorSubcoreMesh(
        core_axis_name="c", subcore_axis_name="s", num_cores=_NC, num_subcores=_NS
    )


@functools.lru_cache(maxsize=None)
def _make_agg(F):
    @functools.partial(
        pl.kernel,
        out_type=jax.ShapeDtypeStruct((_NC, _NP, F), jnp.float32),
        mesh=_mesh(),
        compiler_params=pltpu.CompilerParams(use_tc_tiling_on_sc=False),
        scratch_types=[
            pltpu.VMEM((_NBAT, _K), jnp.int32),
            pltpu.VMEM((_NBAT, _K), jnp.int32),
            pltpu.VMEM((_K, F), jnp.float32),
            pltpu.VMEM_SHARED((_NP, F), jnp.float32),
            pltpu.SemaphoreType.DMA,
        ],
    )
    def agg(src_hbm, dst_hbm, zt_hbm, out_hbm, src_v, dst_v, rows_v, acc, sem):
        c = lax.axis_index("c")
        s = lax.axis_index("s")
        wid = s * _NC + c
        row0 = pl.multiple_of(s * _RPT, _RPT)
        pltpu.sync_copy(src_hbm.at[wid], src_v)
        pltpu.sync_copy(dst_hbm.at[wid], dst_v)
        pltpu.sync_copy(zt_hbm.at[pl.ds(row0, _RPT)], acc.at[pl.ds(row0, _RPT)])
        plsc.subcore_barrier()

        def body(i, carry):
            pltpu.async_copy(zt_hbm.at[src_v.at[i]], rows_v, sem).wait()
            pltpu.sync_copy(rows_v, acc.at[dst_v.at[i]], add=True)
            return carry

        lax.fori_loop(0, _NBAT, body, 0)
        plsc.subcore_barrier()
        pltpu.sync_copy(acc.at[pl.ds(row0, _RPT)], out_hbm.at[c, pl.ds(row0, _RPT)])

    return agg


@functools.lru_cache(maxsize=None)
def _make_deg():
    @functools.partial(
        pl.kernel,
        out_type=jax.ShapeDtypeStruct((_NC, _NP, 16), jnp.float32),
        mesh=_mesh(),
        compiler_params=pltpu.CompilerParams(use_tc_tiling_on_sc=False),
        scratch_types=[
            pltpu.VMEM((_NBAT, _K), jnp.int32),
            pltpu.VMEM((_K, 16), jnp.float32),
            pltpu.VMEM_SHARED((_NP, 16), jnp.float32),
        ],
    )
    def deg_kernel(dst_hbm, ones_hbm, zeros_hbm, out_hbm, dst_v, ones_v, acc):
        c = lax.axis_index("c")
        s = lax.axis_index("s")
        wid = s * _NC + c
        row0 = pl.multiple_of(s * _RPT, _RPT)
        pltpu.sync_copy(dst_hbm.at[wid], dst_v)
        pltpu.sync_copy(ones_hbm, ones_v)
        pltpu.sync_copy(zeros_hbm.at[pl.ds(row0, _RPT)], acc.at[pl.ds(row0, _RPT)])
        plsc.subcore_barrier()

        def body(i, carry):
            pltpu.sync_copy(ones_v, acc.at[dst_v.at[i]], add=True)
            return carry

        lax.fori_loop(0, _NBAT, body, 0)
        plsc.subcore_barrier()
        pltpu.sync_copy(acc.at[pl.ds(row0, _RPT)], out_hbm.at[c, pl.ds(row0, _RPT)])

    return deg_kernel


def _dis_of(degp_ref):
    deg = degp_ref[0, 0:_N, 0:1] + degp_ref[1, 0:_N, 0:1] + 1.0
    return lax.rsqrt(deg)


def _mm_body(x_ref, w_ref, o_ref):
    o_ref[...] = jnp.dot(x_ref[...], w_ref[...], preferred_element_type=jnp.float32)


def _scale_body(degp_ref, y_ref, o_ref):
    o_ref[0:_N, :] = _dis_of(degp_ref) * y_ref[...]
    o_ref[_N:, :] = jnp.zeros((_NP - _N, o_ref.shape[1]), jnp.float32)


def _bn_relu_scale_body(degp_ref, p_ref, zt_ref, g_ref, be_ref, b_ref, o_ref):
    dis = _dis_of(degp_ref)
    agg = p_ref[0, 0:_N, :] + p_ref[1, 0:_N, :] - zt_ref[0:_N, :]
    t = dis * agg + b_ref[...]
    mu = jnp.mean(t, axis=0, keepdims=True)
    var = jnp.mean((t - mu) ** 2, axis=0, keepdims=True)
    h = (t - mu) * lax.rsqrt(var + 1e-5) * g_ref[...] + be_ref[...]
    o_ref[0:_N, :] = dis * jnp.maximum(h, 0.0)
    o_ref[_N:, :] = jnp.zeros((_NP - _N, o_ref.shape[1]), jnp.float32)


def _layer2_body(degp_ref, p_ref, zt_ref, w2_ref, b2_ref, g2_ref, be2_ref,
                 w3_ref, o_ref):
    dis = _dis_of(degp_ref)
    u = dis * (p_ref[0, 0:_N, :] + p_ref[1, 0:_N, :] - zt_ref[0:_N, :])
    t = jnp.dot(u, w2_ref[...], preferred_element_type=jnp.float32) + b2_ref[...]
    mu = jnp.mean(t, axis=0, keepdims=True)
    var = jnp.mean((t - mu) ** 2, axis=0, keepdims=True)
    h = (t - mu) * lax.rsqrt(var + 1e-5) * g2_ref[...] + be2_ref[...]
    h = jnp.maximum(h, 0.0)
    o_ref[0:_N, :] = dis * jnp.dot(h, w3_ref[...], preferred_element_type=jnp.float32)
    o_ref[_N:, :] = jnp.zeros((_NP - _N, o_ref.shape[1]), jnp.float32)


def _pool_body(degp_ref, p_ref, zt_ref, batch_ref, b3_ref, o_ref):
    dis = _dis_of(degp_ref)
    t3 = dis * (p_ref[0, 0:_N, :] + p_ref[1, 0:_N, :] - zt_ref[0:_N, :])
    gids = lax.broadcasted_iota(jnp.int32, (1, 64), 1)
    oh = (batch_ref[...] == gids).astype(jnp.float32)
    dn = (((0,), (0,)), ((), ()))
    sums = lax.dot_general(oh, t3, dn, preferred_element_type=jnp.float32)
    counts = lax.dot_general(oh, jnp.ones((_N, 1), jnp.float32), dn,
                             preferred_element_type=jnp.float32)
    pooled = sums[:, :10] / jnp.maximum(counts, 1.0) + b3_ref[...]
    m = jnp.max(pooled, axis=1, keepdims=True)
    lse = jnp.log(jnp.sum(jnp.exp(pooled - m), axis=1, keepdims=True)) + m
    o_ref[...] = pooled - lse


def _tc(body, out_shape, *args):
    return pl.pallas_call(body, out_shape=out_shape)(*args)


def kernel(x, edge_index, batch, W1, b1, g1, be1, W2, b2, g2, be2, W3, b3):
    f32 = jnp.float32
    src3 = edge_index[0].reshape(_NW, _NBAT, _K)
    dst3 = edge_index[1].reshape(_NW, _NBAT, _K)
    ones_k = jnp.ones((_K, 16), f32)
    zeros_n = jnp.zeros((_NP, 16), f32)

    degp = _make_deg()(dst3, ones_k, zeros_n)
    y1 = _tc(_mm_body, jax.ShapeDtypeStruct((_N, 64), f32), x, W1)
    zt1 = _tc(_scale_body, jax.ShapeDtypeStruct((_NP, 64), f32), degp, y1)

    p1 = _make_agg(64)(src3, dst3, zt1)
    zt2 = _tc(_bn_relu_scale_body, jax.ShapeDtypeStruct((_NP, 64), f32),
              degp, p1, zt1, g1.reshape(1, -1), be1.reshape(1, -1),
              b1.reshape(1, -1))

    p2 = _make_agg(64)(src3, dst3, zt2)
    W3p = jnp.pad(W3, ((0, 0), (0, 16 - W3.shape[1])))
    zt3 = _tc(_layer2_body, jax.ShapeDtypeStruct((_NP, 16), f32),
              degp, p2, zt2, W2, b2.reshape(1, -1), g2.reshape(1, -1),
              be2.reshape(1, -1), W3p)

    p3 = _make_agg(16)(src3, dst3, zt3)
    out = _tc(_pool_body, jax.ShapeDtypeStruct((64, 10), f32),
              degp, p3, zt3, batch.reshape(_N, 1), b3.reshape(1, -1))
    return out


# speedup vs baseline: 36.1066x; 1.3876x over previous
import functools

import jax
import jax.numpy as jnp
from jax import lax
from jax.experimental import pallas as pl
from jax.experimental.pallas import tpu as pltpu
from jax.experimental.pallas import tpu_sc as plsc

_N = 10000
_NP = 10240
_E = 320000
_NC = 2
_NS = 16
_NW = _NC * _NS
_EPW = _E // _NW
_K = 125
_NBAT = _EPW // _K
_RPT = _NP // _NS


@functools.lru_cache(maxsize=None)
def _mesh():
    return plsc.VectorSubcoreMesh(
        core_axis_name="c", subcore_axis_name="s", num_cores=_NC, num_subcores=_NS
    )


@functools.lru_cache(maxsize=None)
def _make_agg(F):
    @functools.partial(
        pl.kernel,
        out_type=jax.ShapeDtypeStruct((_NC, _NP, F), jnp.float32),
        mesh=_mesh(),
        compiler_params=pltpu.CompilerParams(use_tc_tiling_on_sc=False),
        scratch_types=[
            pltpu.VMEM((_NBAT, _K), jnp.int32),
            pltpu.VMEM((_NBAT, _K), jnp.int32),
            pltpu.VMEM((_K, F), jnp.float32),
            pltpu.VMEM((_K, F), jnp.float32),
            pltpu.VMEM_SHARED((_NP, F), jnp.float32),
            pltpu.SemaphoreType.DMA,
            pltpu.SemaphoreType.DMA,
            pltpu.SemaphoreType.DMA,
            pltpu.SemaphoreType.DMA,
        ],
    )
    def agg(src_hbm, dst_hbm, zt_hbm, out_hbm, src_v, dst_v, rows0, rows1,
            acc, gsem0, gsem1, ssem0, ssem1):
        c = lax.axis_index("c")
        s = lax.axis_index("s")
        wid = s * _NC + c
        row0 = pl.multiple_of(s * _RPT, _RPT)
        pltpu.sync_copy(src_hbm.at[wid], src_v)
        pltpu.sync_copy(dst_hbm.at[wid], dst_v)
        pltpu.sync_copy(zt_hbm.at[pl.ds(row0, _RPT)], acc.at[pl.ds(row0, _RPT)])
        plsc.subcore_barrier()

        rows = (rows0, rows1)
        gsem = (gsem0, gsem1)
        ssem = (ssem0, ssem1)

        pltpu.async_copy(zt_hbm.at[src_v.at[0]], rows0, gsem0)
        pltpu.async_copy(zt_hbm.at[src_v.at[1]], rows1, gsem1)

        def body(j, carry):
            for b in range(2):
                i = 2 * j + b
                pltpu.make_async_copy(zt_hbm.at[src_v.at[i]], rows[b],
                                      gsem[b]).wait()
                pltpu.async_copy(rows[b], acc.at[dst_v.at[i]], ssem[b],
                                 add=True)

                @pl.when(i + 2 < _NBAT)
                def _():
                    pltpu.make_async_copy(rows[b], acc.at[dst_v.at[i]],
                                          ssem[b]).wait()
                    pltpu.async_copy(zt_hbm.at[src_v.at[i + 2]], rows[b],
                                     gsem[b])

            return carry

        lax.fori_loop(0, _NBAT // 2, body, 0)
        for b in range(2):
            i = _NBAT - 2 + b
            pltpu.make_async_copy(rows[b], acc.at[dst_v.at[i]], ssem[b]).wait()
        plsc.subcore_barrier()
        pltpu.sync_copy(acc.at[pl.ds(row0, _RPT)], out_hbm.at[c, pl.ds(row0, _RPT)])

    return agg


@functools.lru_cache(maxsize=None)
def _make_deg():
    @functools.partial(
        pl.kernel,
        out_type=jax.ShapeDtypeStruct((_NC, _NP, 16), jnp.float32),
        mesh=_mesh(),
        compiler_params=pltpu.CompilerParams(use_tc_tiling_on_sc=False),
        scratch_types=[
            pltpu.VMEM((_NBAT, _K), jnp.int32),
            pltpu.VMEM((_K, 16), jnp.float32),
            pltpu.VMEM_SHARED((_NP, 16), jnp.float32),
        ],
    )
    def deg_kernel(dst_hbm, ones_hbm, zeros_hbm, out_hbm, dst_v, ones_v, acc):
        c = lax.axis_index("c")
        s = lax.axis_index("s")
        wid = s * _NC + c
        row0 = pl.multiple_of(s * _RPT, _RPT)
        pltpu.sync_copy(dst_hbm.at[wid], dst_v)
        pltpu.sync_copy(ones_hbm, ones_v)
        pltpu.sync_copy(zeros_hbm.at[pl.ds(row0, _RPT)], acc.at[pl.ds(row0, _RPT)])
        plsc.subcore_barrier()

        def body(i, carry):
            pltpu.sync_copy(ones_v, acc.at[dst_v.at[i]], add=True)
            return carry

        lax.fori_loop(0, _NBAT, body, 0)
        plsc.subcore_barrier()
        pltpu.sync_copy(acc.at[pl.ds(row0, _RPT)], out_hbm.at[c, pl.ds(row0, _RPT)])

    return deg_kernel


def _dis_of(degp_ref):
    deg = degp_ref[0, 0:_N, 0:1] + degp_ref[1, 0:_N, 0:1] + 1.0
    return lax.rsqrt(deg)


def _mm_body(x_ref, w_ref, o_ref):
    o_ref[...] = jnp.dot(x_ref[...], w_ref[...], preferred_element_type=jnp.float32)


def _scale_body(degp_ref, y_ref, o_ref):
    o_ref[0:_N, :] = _dis_of(degp_ref) * y_ref[...]
    o_ref[_N:, :] = jnp.zeros((_NP - _N, o_ref.shape[1]), jnp.float32)


def _bn_relu_scale_body(degp_ref, p_ref, zt_ref, g_ref, be_ref, b_ref, o_ref):
    dis = _dis_of(degp_ref)
    agg = p_ref[0, 0:_N, :] + p_ref[1, 0:_N, :] - zt_ref[0:_N, :]
    t = dis * agg + b_ref[...]
    mu = jnp.mean(t, axis=0, keepdims=True)
    var = jnp.mean((t - mu) ** 2, axis=0, keepdims=True)
    h = (t - mu) * lax.rsqrt(var + 1e-5) * g_ref[...] + be_ref[...]
    o_ref[0:_N, :] = dis * jnp.maximum(h, 0.0)
    o_ref[_N:, :] = jnp.zeros((_NP - _N, o_ref.shape[1]), jnp.float32)


def _layer2_body(degp_ref, p_ref, zt_ref, w2_ref, b2_ref, g2_ref, be2_ref,
                 w3_ref, o_ref):
    dis = _dis_of(degp_ref)
    u = dis * (p_ref[0, 0:_N, :] + p_ref[1, 0:_N, :] - zt_ref[0:_N, :])
    t = jnp.dot(u, w2_ref[...], preferred_element_type=jnp.float32) + b2_ref[...]
    mu = jnp.mean(t, axis=0, keepdims=True)
    var = jnp.mean((t - mu) ** 2, axis=0, keepdims=True)
    h = (t - mu) * lax.rsqrt(var + 1e-5) * g2_ref[...] + be2_ref[...]
    h = jnp.maximum(h, 0.0)
    o_ref[0:_N, :] = dis * jnp.dot(h, w3_ref[...], preferred_element_type=jnp.float32)
    o_ref[_N:, :] = jnp.zeros((_NP - _N, o_ref.shape[1]), jnp.float32)


def _pool_body(degp_ref, p_ref, zt_ref, batch_ref, b3_ref, o_ref):
    dis = _dis_of(degp_ref)
    t3 = dis * (p_ref[0, 0:_N, :] + p_ref[1, 0:_N, :] - zt_ref[0:_N, :])
    gids = lax.broadcasted_iota(jnp.int32, (1, 64), 1)
    oh = (batch_ref[...] == gids).astype(jnp.float32)
    dn = (((0,), (0,)), ((), ()))
    sums = lax.dot_general(oh, t3, dn, preferred_element_type=jnp.float32)
    counts = lax.dot_general(oh, jnp.ones((_N, 1), jnp.float32), dn,
                             preferred_element_type=jnp.float32)
    pooled = sums[:, :10] / jnp.maximum(counts, 1.0) + b3_ref[...]
    m = jnp.max(pooled, axis=1, keepdims=True)
    lse = jnp.log(jnp.sum(jnp.exp(pooled - m), axis=1, keepdims=True)) + m
    o_ref[...] = pooled - lse


def _tc(body, out_shape, *args):
    return pl.pallas_call(body, out_shape=out_shape)(*args)


def kernel(x, edge_index, batch, W1, b1, g1, be1, W2, b2, g2, be2, W3, b3):
    f32 = jnp.float32
    src3 = edge_index[0].reshape(_NW, _NBAT, _K)
    dst3 = edge_index[1].reshape(_NW, _NBAT, _K)
    ones_k = jnp.ones((_K, 16), f32)
    zeros_n = jnp.zeros((_NP, 16), f32)

    degp = _make_deg()(dst3, ones_k, zeros_n)
    y1 = _tc(_mm_body, jax.ShapeDtypeStruct((_N, 64), f32), x, W1)
    zt1 = _tc(_scale_body, jax.ShapeDtypeStruct((_NP, 64), f32), degp, y1)

    p1 = _make_agg(64)(src3, dst3, zt1)
    zt2 = _tc(_bn_relu_scale_body, jax.ShapeDtypeStruct((_NP, 64), f32),
              degp, p1, zt1, g1.reshape(1, -1), be1.reshape(1, -1),
              b1.reshape(1, -1))

    p2 = _make_agg(64)(src3, dst3, zt2)
    W3p = jnp.pad(W3, ((0, 0), (0, 16 - W3.shape[1])))
    zt3 = _tc(_layer2_body, jax.ShapeDtypeStruct((_NP, 16), f32),
              degp, p2, zt2, W2, b2.reshape(1, -1), g2.reshape(1, -1),
              be2.reshape(1, -1), W3p)

    p3 = _make_agg(16)(src3, dst3, zt3)
    out = _tc(_pool_body, jax.ShapeDtypeStruct((64, 10), f32),
              degp, p3, zt3, batch.reshape(_N, 1), b3.reshape(1, -1))
    return out


# speedup vs baseline: 42.6576x; 1.1814x over previous
import functools

import jax
import jax.numpy as jnp
from jax import lax
from jax.experimental import pallas as pl
from jax.experimental.pallas import tpu as pltpu
from jax.experimental.pallas import tpu_sc as plsc

_N = 10000
_NP = 10240
_E = 320000
_NC = 2
_NS = 16
_NW = _NC * _NS
_EPW = _E // _NW
_K = 125
_NBAT = _EPW // _K
_RPT = _NP // _NS
_NBUF = 4


@functools.lru_cache(maxsize=None)
def _mesh():
    return plsc.VectorSubcoreMesh(
        core_axis_name="c", subcore_axis_name="s", num_cores=_NC, num_subcores=_NS
    )


@functools.lru_cache(maxsize=None)
def _make_agg(F):
    @functools.partial(
        pl.kernel,
        out_type=jax.ShapeDtypeStruct((_NC, _NP, F), jnp.float32),
        mesh=_mesh(),
        compiler_params=pltpu.CompilerParams(use_tc_tiling_on_sc=False),
        scratch_types=[
            pltpu.VMEM((_NBAT, _K), jnp.int32),
            pltpu.VMEM((_NBAT, _K), jnp.int32),
            [pltpu.VMEM((_K, F), jnp.float32)] * _NBUF,
            [pltpu.SemaphoreType.DMA] * _NBUF,
            [pltpu.SemaphoreType.DMA] * _NBUF,
            pltpu.VMEM_SHARED((_NP, F), jnp.float32),
        ],
    )
    def agg(src_hbm, dst_hbm, zt_hbm, out_hbm, src_v, dst_v, rows, gsem,
            ssem, acc):
        c = lax.axis_index("c")
        s = lax.axis_index("s")
        wid = s * _NC + c
        row0 = pl.multiple_of(s * _RPT, _RPT)
        pltpu.sync_copy(src_hbm.at[wid], src_v)
        pltpu.sync_copy(dst_hbm.at[wid], dst_v)
        pltpu.sync_copy(zt_hbm.at[pl.ds(row0, _RPT)], acc.at[pl.ds(row0, _RPT)])
        plsc.subcore_barrier()

        for b in range(_NBUF):
            pltpu.async_copy(zt_hbm.at[src_v.at[b]], rows[b], gsem[b])

        def body(j, carry):
            for b in range(_NBUF):
                i = _NBUF * j + b
                pltpu.make_async_copy(zt_hbm.at[src_v.at[i]], rows[b],
                                      gsem[b]).wait()
                pltpu.async_copy(rows[b], acc.at[dst_v.at[i]], ssem[b],
                                 add=True)

                @pl.when(i + _NBUF < _NBAT)
                def _():
                    pltpu.make_async_copy(rows[b], acc.at[dst_v.at[i]],
                                          ssem[b]).wait()
                    pltpu.async_copy(zt_hbm.at[src_v.at[i + _NBUF]], rows[b],
                                     gsem[b])

            return carry

        lax.fori_loop(0, _NBAT // _NBUF, body, 0)
        for b in range(_NBUF):
            i = _NBAT - _NBUF + b
            pltpu.make_async_copy(rows[b], acc.at[dst_v.at[i]], ssem[b]).wait()
        plsc.subcore_barrier()
        pltpu.sync_copy(acc.at[pl.ds(row0, _RPT)], out_hbm.at[c, pl.ds(row0, _RPT)])

    return agg


@functools.lru_cache(maxsize=None)
def _make_deg():
    @functools.partial(
        pl.kernel,
        out_type=jax.ShapeDtypeStruct((_NC, _NP, 16), jnp.float32),
        mesh=_mesh(),
        compiler_params=pltpu.CompilerParams(use_tc_tiling_on_sc=False),
        scratch_types=[
            pltpu.VMEM((_NBAT, _K), jnp.int32),
            pltpu.VMEM((_K, 16), jnp.float32),
            pltpu.VMEM_SHARED((_NP, 16), jnp.float32),
            pltpu.SemaphoreType.DMA,
        ],
    )
    def deg_kernel(dst_hbm, ones_hbm, zeros_hbm, out_hbm, dst_v, ones_v, acc,
                   sem):
        c = lax.axis_index("c")
        s = lax.axis_index("s")
        wid = s * _NC + c
        row0 = pl.multiple_of(s * _RPT, _RPT)
        pltpu.sync_copy(dst_hbm.at[wid], dst_v)
        pltpu.sync_copy(ones_hbm, ones_v)
        pltpu.sync_copy(zeros_hbm.at[pl.ds(row0, _RPT)], acc.at[pl.ds(row0, _RPT)])
        plsc.subcore_barrier()

        def body(j, carry):
            for t in range(16):
                pltpu.async_copy(ones_v, acc.at[dst_v.at[16 * j + t]], sem,
                                 add=True)
            for t in range(16):
                pltpu.make_async_copy(ones_v, acc.at[dst_v.at[16 * j + t]],
                                      sem).wait()
            return carry

        lax.fori_loop(0, _NBAT // 16, body, 0)
        plsc.subcore_barrier()
        pltpu.sync_copy(acc.at[pl.ds(row0, _RPT)], out_hbm.at[c, pl.ds(row0, _RPT)])

    return deg_kernel


def _dis_of(degp_ref):
    deg = degp_ref[0, 0:_N, 0:1] + degp_ref[1, 0:_N, 0:1] + 1.0
    return lax.rsqrt(deg)


def _mm_scale_body(degp_ref, x_ref, w_ref, o_ref):
    y = jnp.dot(x_ref[...], w_ref[...], preferred_element_type=jnp.float32)
    o_ref[0:_N, :] = _dis_of(degp_ref) * y
    o_ref[_N:, :] = jnp.zeros((_NP - _N, o_ref.shape[1]), jnp.float32)


def _bn_relu_scale_body(degp_ref, p_ref, zt_ref, g_ref, be_ref, b_ref, o_ref):
    dis = _dis_of(degp_ref)
    agg = p_ref[0, 0:_N, :] + p_ref[1, 0:_N, :] - zt_ref[0:_N, :]
    t = dis * agg + b_ref[...]
    mu = jnp.mean(t, axis=0, keepdims=True)
    var = jnp.mean((t - mu) ** 2, axis=0, keepdims=True)
    h = (t - mu) * lax.rsqrt(var + 1e-5) * g_ref[...] + be_ref[...]
    o_ref[0:_N, :] = dis * jnp.maximum(h, 0.0)
    o_ref[_N:, :] = jnp.zeros((_NP - _N, o_ref.shape[1]), jnp.float32)


def _layer2_body(degp_ref, p_ref, zt_ref, w2_ref, b2_ref, g2_ref, be2_ref,
                 w3_ref, o_ref):
    dis = _dis_of(degp_ref)
    u = dis * (p_ref[0, 0:_N, :] + p_ref[1, 0:_N, :] - zt_ref[0:_N, :])
    t = jnp.dot(u, w2_ref[...], preferred_element_type=jnp.float32) + b2_ref[...]
    mu = jnp.mean(t, axis=0, keepdims=True)
    var = jnp.mean((t - mu) ** 2, axis=0, keepdims=True)
    h = (t - mu) * lax.rsqrt(var + 1e-5) * g2_ref[...] + be2_ref[...]
    h = jnp.maximum(h, 0.0)
    o_ref[0:_N, :] = dis * jnp.dot(h, w3_ref[...], preferred_element_type=jnp.float32)
    o_ref[_N:, :] = jnp.zeros((_NP - _N, o_ref.shape[1]), jnp.float32)


def _pool_body(degp_ref, p_ref, zt_ref, batch_ref, b3_ref, o_ref):
    dis = _dis_of(degp_ref)
    t3 = dis * (p_ref[0, 0:_N, :] + p_ref[1, 0:_N, :] - zt_ref[0:_N, :])
    gids = lax.broadcasted_iota(jnp.int32, (1, 64), 1)
    oh = (batch_ref[...] == gids).astype(jnp.float32)
    dn = (((0,), (0,)), ((), ()))
    sums = lax.dot_general(oh, t3, dn, preferred_element_type=jnp.float32)
    counts = lax.dot_general(oh, jnp.ones((_N, 1), jnp.float32), dn,
                             preferred_element_type=jnp.float32)
    pooled = sums[:, :10] / jnp.maximum(counts, 1.0) + b3_ref[...]
    m = jnp.max(pooled, axis=1, keepdims=True)
    lse = jnp.log(jnp.sum(jnp.exp(pooled - m), axis=1, keepdims=True)) + m
    o_ref[...] = pooled - lse


def _tc(body, out_shape, *args):
    return pl.pallas_call(body, out_shape=out_shape)(*args)


def kernel(x, edge_index, batch, W1, b1, g1, be1, W2, b2, g2, be2, W3, b3):
    f32 = jnp.float32
    src3 = edge_index[0].reshape(_NW, _NBAT, _K)
    dst3 = edge_index[1].reshape(_NW, _NBAT, _K)
    ones_k = jnp.ones((_K, 16), f32)
    zeros_n = jnp.zeros((_NP, 16), f32)

    degp = _make_deg()(dst3, ones_k, zeros_n)
    zt1 = _tc(_mm_scale_body, jax.ShapeDtypeStruct((_NP, 64), f32),
              degp, x, W1)

    p1 = _make_agg(64)(src3, dst3, zt1)
    zt2 = _tc(_bn_relu_scale_body, jax.ShapeDtypeStruct((_NP, 64), f32),
              degp, p1, zt1, g1.reshape(1, -1), be1.reshape(1, -1),
              b1.reshape(1, -1))

    p2 = _make_agg(64)(src3, dst3, zt2)
    W3p = jnp.pad(W3, ((0, 0), (0, 16 - W3.shape[1])))
    zt3 = _tc(_layer2_body, jax.ShapeDtypeStruct((_NP, 16), f32),
              degp, p2, zt2, W2, b2.reshape(1, -1), g2.reshape(1, -1),
              be2.reshape(1, -1), W3p)

    p3 = _make_agg(16)(src3, dst3, zt3)
    out = _tc(_pool_body, jax.ShapeDtypeStruct((64, 10), f32),
              degp, p3, zt3, batch.reshape(_N, 1), b3.reshape(1, -1))
    return out


# speedup vs baseline: 43.7112x; 1.0247x over previous
import functools

import jax
import jax.numpy as jnp
from jax import lax
from jax.experimental import pallas as pl
from jax.experimental.pallas import tpu as pltpu
from jax.experimental.pallas import tpu_sc as plsc

_N = 10000
_NP = 10240
_E = 320000
_NC = 2
_NS = 16
_NW = _NC * _NS
_EPW = _E // _NW
_K = 125
_NBAT = _EPW // _K
_RPT = _NP // _NS
_NBUF = 8


@functools.lru_cache(maxsize=None)
def _mesh():
    return plsc.VectorSubcoreMesh(
        core_axis_name="c", subcore_axis_name="s", num_cores=_NC, num_subcores=_NS
    )


@functools.lru_cache(maxsize=None)
def _make_agg(F):
    @functools.partial(
        pl.kernel,
        out_type=jax.ShapeDtypeStruct((_NC, _NP, F), jnp.float32),
        mesh=_mesh(),
        compiler_params=pltpu.CompilerParams(use_tc_tiling_on_sc=False),
        scratch_types=[
            pltpu.VMEM((_NBAT, _K), jnp.int32),
            pltpu.VMEM((_NBAT, _K), jnp.int32),
            [pltpu.VMEM((_K, F), jnp.float32)] * _NBUF,
            [pltpu.SemaphoreType.DMA] * _NBUF,
            [pltpu.SemaphoreType.DMA] * _NBUF,
            pltpu.VMEM_SHARED((_NP, F), jnp.float32),
        ],
    )
    def agg(src_hbm, dst_hbm, zt_hbm, out_hbm, src_v, dst_v, rows, gsem,
            ssem, acc):
        c = lax.axis_index("c")
        s = lax.axis_index("s")
        wid = s * _NC + c
        row0 = pl.multiple_of(s * _RPT, _RPT)
        pltpu.sync_copy(src_hbm.at[wid], src_v)
        pltpu.sync_copy(dst_hbm.at[wid], dst_v)
        pltpu.sync_copy(zt_hbm.at[pl.ds(row0, _RPT)], acc.at[pl.ds(row0, _RPT)])
        plsc.subcore_barrier()

        for b in range(_NBUF):
            pltpu.async_copy(zt_hbm.at[src_v.at[b]], rows[b], gsem[b])

        def body(j, carry):
            for b in range(_NBUF):
                i = _NBUF * j + b
                pltpu.make_async_copy(zt_hbm.at[src_v.at[i]], rows[b],
                                      gsem[b]).wait()
                pltpu.async_copy(rows[b], acc.at[dst_v.at[i]], ssem[b],
                                 add=True)

                @pl.when(i + _NBUF < _NBAT)
                def _():
                    pltpu.make_async_copy(rows[b], acc.at[dst_v.at[i]],
                                          ssem[b]).wait()
                    pltpu.async_copy(zt_hbm.at[src_v.at[i + _NBUF]], rows[b],
                                     gsem[b])

            return carry

        lax.fori_loop(0, _NBAT // _NBUF, body, 0)
        for b in range(_NBUF):
            i = _NBAT - _NBUF + b
            pltpu.make_async_copy(rows[b], acc.at[dst_v.at[i]], ssem[b]).wait()
        plsc.subcore_barrier()
        pltpu.sync_copy(acc.at[pl.ds(row0, _RPT)], out_hbm.at[c, pl.ds(row0, _RPT)])

    return agg


@functools.lru_cache(maxsize=None)
def _make_deg():
    @functools.partial(
        pl.kernel,
        out_type=jax.ShapeDtypeStruct((_NC, _NP, 16), jnp.float32),
        mesh=_mesh(),
        compiler_params=pltpu.CompilerParams(use_tc_tiling_on_sc=False),
        scratch_types=[
            pltpu.VMEM((_NBAT, _K), jnp.int32),
            pltpu.VMEM((_K, 16), jnp.float32),
            pltpu.VMEM_SHARED((_NP, 16), jnp.float32),
            pltpu.SemaphoreType.DMA,
        ],
    )
    def deg_kernel(dst_hbm, ones_hbm, zeros_hbm, out_hbm, dst_v, ones_v, acc,
                   sem):
        c = lax.axis_index("c")
        s = lax.axis_index("s")
        wid = s * _NC + c
        row0 = pl.multiple_of(s * _RPT, _RPT)
        pltpu.sync_copy(dst_hbm.at[wid], dst_v)
        pltpu.sync_copy(ones_hbm, ones_v)
        pltpu.sync_copy(zeros_hbm.at[pl.ds(row0, _RPT)], acc.at[pl.ds(row0, _RPT)])
        plsc.subcore_barrier()

        def body(j, carry):
            for t in range(16):
                pltpu.async_copy(ones_v, acc.at[dst_v.at[16 * j + t]], sem,
                                 add=True)
            for t in range(16):
                pltpu.make_async_copy(ones_v, acc.at[dst_v.at[16 * j + t]],
                                      sem).wait()
            return carry

        lax.fori_loop(0, _NBAT // 16, body, 0)
        plsc.subcore_barrier()
        pltpu.sync_copy(acc.at[pl.ds(row0, _RPT)], out_hbm.at[c, pl.ds(row0, _RPT)])

    return deg_kernel


def _dis_of(degp_ref):
    deg = degp_ref[0, 0:_N, 0:1] + degp_ref[1, 0:_N, 0:1] + 1.0
    return lax.rsqrt(deg)


def _mm_scale_body(degp_ref, x_ref, w_ref, o_ref):
    y = jnp.dot(x_ref[...], w_ref[...], preferred_element_type=jnp.float32)
    o_ref[0:_N, :] = _dis_of(degp_ref) * y
    o_ref[_N:, :] = jnp.zeros((_NP - _N, o_ref.shape[1]), jnp.float32)


def _bn_relu_scale_body(degp_ref, p_ref, zt_ref, g_ref, be_ref, b_ref, o_ref):
    dis = _dis_of(degp_ref)
    agg = p_ref[0, 0:_N, :] + p_ref[1, 0:_N, :] - zt_ref[0:_N, :]
    t = dis * agg + b_ref[...]
    mu = jnp.mean(t, axis=0, keepdims=True)
    var = jnp.mean((t - mu) ** 2, axis=0, keepdims=True)
    h = (t - mu) * lax.rsqrt(var + 1e-5) * g_ref[...] + be_ref[...]
    o_ref[0:_N, :] = dis * jnp.maximum(h, 0.0)
    o_ref[_N:, :] = jnp.zeros((_NP - _N, o_ref.shape[1]), jnp.float32)


def _layer2_body(degp_ref, p_ref, zt_ref, w2_ref, b2_ref, g2_ref, be2_ref,
                 w3_ref, o_ref):
    dis = _dis_of(degp_ref)
    u = dis * (p_ref[0, 0:_N, :] + p_ref[1, 0:_N, :] - zt_ref[0:_N, :])
    t = jnp.dot(u, w2_ref[...], preferred_element_type=jnp.float32) + b2_ref[...]
    mu = jnp.mean(t, axis=0, keepdims=True)
    var = jnp.mean((t - mu) ** 2, axis=0, keepdims=True)
    h = (t - mu) * lax.rsqrt(var + 1e-5) * g2_ref[...] + be2_ref[...]
    h = jnp.maximum(h, 0.0)
    o_ref[0:_N, :] = dis * jnp.dot(h, w3_ref[...], preferred_element_type=jnp.float32)
    o_ref[_N:, :] = jnp.zeros((_NP - _N, o_ref.shape[1]), jnp.float32)


def _pool_body(degp_ref, p_ref, zt_ref, batch_ref, b3_ref, o_ref):
    dis = _dis_of(degp_ref)
    t3 = dis * (p_ref[0, 0:_N, :] + p_ref[1, 0:_N, :] - zt_ref[0:_N, :])
    gids = lax.broadcasted_iota(jnp.int32, (1, 64), 1)
    oh = (batch_ref[...] == gids).astype(jnp.float32)
    dn = (((0,), (0,)), ((), ()))
    sums = lax.dot_general(oh, t3, dn, preferred_element_type=jnp.float32)
    counts = lax.dot_general(oh, jnp.ones((_N, 1), jnp.float32), dn,
                             preferred_element_type=jnp.float32)
    pooled = sums[:, :10] / jnp.maximum(counts, 1.0) + b3_ref[...]
    m = jnp.max(pooled, axis=1, keepdims=True)
    lse = jnp.log(jnp.sum(jnp.exp(pooled - m), axis=1, keepdims=True)) + m
    o_ref[...] = pooled - lse


def _tc(body, out_shape, *args):
    return pl.pallas_call(body, out_shape=out_shape)(*args)


def kernel(x, edge_index, batch, W1, b1, g1, be1, W2, b2, g2, be2, W3, b3):
    f32 = jnp.float32
    src3 = edge_index[0].reshape(_NW, _NBAT, _K)
    dst3 = edge_index[1].reshape(_NW, _NBAT, _K)
    ones_k = jnp.ones((_K, 16), f32)
    zeros_n = jnp.zeros((_NP, 16), f32)

    degp = _make_deg()(dst3, ones_k, zeros_n)
    zt1 = _tc(_mm_scale_body, jax.ShapeDtypeStruct((_NP, 64), f32),
              degp, x, W1)

    p1 = _make_agg(64)(src3, dst3, zt1)
    zt2 = _tc(_bn_relu_scale_body, jax.ShapeDtypeStruct((_NP, 64), f32),
              degp, p1, zt1, g1.reshape(1, -1), be1.reshape(1, -1),
              b1.reshape(1, -1))

    p2 = _make_agg(64)(src3, dst3, zt2)
    W3p = jnp.pad(W3, ((0, 0), (0, 16 - W3.shape[1])))
    zt3 = _tc(_layer2_body, jax.ShapeDtypeStruct((_NP, 16), f32),
              degp, p2, zt2, W2, b2.reshape(1, -1), g2.reshape(1, -1),
              be2.reshape(1, -1), W3p)

    p3 = _make_agg(16)(src3, dst3, zt3)
    out = _tc(_pool_body, jax.ShapeDtypeStruct((64, 10), f32),
              degp, p3, zt3, batch.reshape(_N, 1), b3.reshape(1, -1))
    return out


# speedup vs baseline: 45.3164x; 1.0367x over previous
import functools

import jax
import jax.numpy as jnp
from jax import lax
from jax.experimental import pallas as pl
from jax.experimental.pallas import tpu as pltpu
from jax.experimental.pallas import tpu_sc as plsc

_N = 10000
_NP = 10240
_E = 320000
_NC = 2
_NS = 16
_NW = _NC * _NS
_EPW = _E // _NW
_K = 125
_NBAT = _EPW // _K
_RPT = _NP // _NS
_NBUF = 8


@functools.lru_cache(maxsize=None)
def _mesh():
    return plsc.VectorSubcoreMesh(
        core_axis_name="c", subcore_axis_name="s", num_cores=_NC, num_subcores=_NS
    )


@functools.lru_cache(maxsize=None)
def _make_agg(F):
    @functools.partial(
        pl.kernel,
        out_type=jax.ShapeDtypeStruct((_NC, _NP, F), jnp.float32),
        mesh=_mesh(),
        compiler_params=pltpu.CompilerParams(use_tc_tiling_on_sc=False),
        scratch_types=[
            pltpu.VMEM((_NBAT, _K), jnp.int32),
            pltpu.VMEM((_NBAT, _K), jnp.int32),
            [pltpu.VMEM((_K, F), jnp.float32)] * _NBUF,
            [pltpu.SemaphoreType.DMA] * _NBUF,
            [pltpu.SemaphoreType.DMA] * _NBUF,
            pltpu.VMEM_SHARED((_NP, F), jnp.float32),
        ],
    )
    def agg(ei_hbm, zt_hbm, out_hbm, src_v, dst_v, rows, gsem, ssem, acc):
        c = lax.axis_index("c")
        s = lax.axis_index("s")
        wid = s * _NC + c
        row0 = pl.multiple_of(s * _RPT, _RPT)
        pltpu.sync_copy(ei_hbm.at[wid], src_v)
        pltpu.sync_copy(ei_hbm.at[_NW + wid], dst_v)
        pltpu.sync_copy(zt_hbm.at[pl.ds(row0, _RPT)], acc.at[pl.ds(row0, _RPT)])
        plsc.subcore_barrier()

        for b in range(_NBUF):
            pltpu.async_copy(zt_hbm.at[src_v.at[b]], rows[b], gsem[b])

        def body(j, carry):
            for b in range(_NBUF):
                i = _NBUF * j + b
                pltpu.make_async_copy(zt_hbm.at[src_v.at[i]], rows[b],
                                      gsem[b]).wait()
                pltpu.async_copy(rows[b], acc.at[dst_v.at[i]], ssem[b],
                                 add=True)

                @pl.when(i + _NBUF < _NBAT)
                def _():
                    pltpu.make_async_copy(rows[b], acc.at[dst_v.at[i]],
                                          ssem[b]).wait()
                    pltpu.async_copy(zt_hbm.at[src_v.at[i + _NBUF]], rows[b],
                                     gsem[b])

            return carry

        lax.fori_loop(0, _NBAT // _NBUF, body, 0)
        for b in range(_NBUF):
            i = _NBAT - _NBUF + b
            pltpu.make_async_copy(rows[b], acc.at[dst_v.at[i]], ssem[b]).wait()
        plsc.subcore_barrier()
        pltpu.sync_copy(acc.at[pl.ds(row0, _RPT)], out_hbm.at[c, pl.ds(row0, _RPT)])

    return agg


@functools.lru_cache(maxsize=None)
def _make_deg():
    @functools.partial(
        pl.kernel,
        out_type=jax.ShapeDtypeStruct((_NC, _NP, 16), jnp.float32),
        mesh=_mesh(),
        compiler_params=pltpu.CompilerParams(use_tc_tiling_on_sc=False),
        scratch_types=[
            pltpu.VMEM((_NBAT, _K), jnp.int32),
            pltpu.VMEM((_K, 16), jnp.float32),
            pltpu.VMEM_SHARED((_NP, 16), jnp.float32),
            pltpu.SemaphoreType.DMA,
        ],
    )
    def deg_kernel(ei_hbm, ones_hbm, zeros_hbm, out_hbm, dst_v, ones_v, acc,
                   sem):
        c = lax.axis_index("c")
        s = lax.axis_index("s")
        wid = s * _NC + c
        row0 = pl.multiple_of(s * _RPT, _RPT)
        pltpu.sync_copy(ei_hbm.at[_NW + wid], dst_v)
        pltpu.sync_copy(ones_hbm, ones_v)
        pltpu.sync_copy(zeros_hbm.at[pl.ds(row0, _RPT)], acc.at[pl.ds(row0, _RPT)])
        plsc.subcore_barrier()

        def body(j, carry):
            for t in range(16):
                pltpu.async_copy(ones_v, acc.at[dst_v.at[16 * j + t]], sem,
                                 add=True)
            for t in range(16):
                pltpu.make_async_copy(ones_v, acc.at[dst_v.at[16 * j + t]],
                                      sem).wait()
            return carry

        lax.fori_loop(0, _NBAT // 16, body, 0)
        plsc.subcore_barrier()
        pltpu.sync_copy(acc.at[pl.ds(row0, _RPT)], out_hbm.at[c, pl.ds(row0, _RPT)])

    return deg_kernel


def _dis_of(degp_ref):
    deg = degp_ref[0, 0:_N, 0:1] + degp_ref[1, 0:_N, 0:1] + 1.0
    return lax.rsqrt(deg)


def _mm_scale_body(degp_ref, x_ref, w_ref, o_ref):
    y = jnp.dot(x_ref[...], w_ref[...], preferred_element_type=jnp.float32)
    o_ref[0:_N, :] = _dis_of(degp_ref) * y
    o_ref[_N:, :] = jnp.zeros((_NP - _N, o_ref.shape[1]), jnp.float32)


def _bn_relu_scale_body(degp_ref, p_ref, zt_ref, g_ref, be_ref, b_ref, o_ref):
    dis = _dis_of(degp_ref)
    agg = p_ref[0, 0:_N, :] + p_ref[1, 0:_N, :] - zt_ref[0:_N, :]
    t = dis * agg + b_ref[...]
    mu = jnp.mean(t, axis=0, keepdims=True)
    var = jnp.mean((t - mu) ** 2, axis=0, keepdims=True)
    h = (t - mu) * lax.rsqrt(var + 1e-5) * g_ref[...] + be_ref[...]
    o_ref[0:_N, :] = dis * jnp.maximum(h, 0.0)
    o_ref[_N:, :] = jnp.zeros((_NP - _N, o_ref.shape[1]), jnp.float32)


def _layer2_body(degp_ref, p_ref, zt_ref, w2_ref, b2_ref, g2_ref, be2_ref,
                 w3_ref, o_ref):
    dis = _dis_of(degp_ref)
    u = dis * (p_ref[0, 0:_N, :] + p_ref[1, 0:_N, :] - zt_ref[0:_N, :])
    t = jnp.dot(u, w2_ref[...], preferred_element_type=jnp.float32) + b2_ref[...]
    mu = jnp.mean(t, axis=0, keepdims=True)
    var = jnp.mean((t - mu) ** 2, axis=0, keepdims=True)
    h = (t - mu) * lax.rsqrt(var + 1e-5) * g2_ref[...] + be2_ref[...]
    h = jnp.maximum(h, 0.0)
    o_ref[0:_N, :] = dis * jnp.dot(h, w3_ref[...], preferred_element_type=jnp.float32)
    o_ref[_N:, :] = jnp.zeros((_NP - _N, o_ref.shape[1]), jnp.float32)


def _pool_body(degp_ref, p_ref, zt_ref, batch_ref, b3_ref, o_ref):
    dis = _dis_of(degp_ref)
    t3 = dis * (p_ref[0, 0:_N, :] + p_ref[1, 0:_N, :] - zt_ref[0:_N, :])
    gids = lax.broadcasted_iota(jnp.int32, (1, 64), 1)
    oh = (batch_ref[...] == gids).astype(jnp.float32)
    dn = (((0,), (0,)), ((), ()))
    sums = lax.dot_general(oh, t3, dn, preferred_element_type=jnp.float32)
    counts = lax.dot_general(oh, jnp.ones((_N, 1), jnp.float32), dn,
                             preferred_element_type=jnp.float32)
    pooled = sums[:, :10] / jnp.maximum(counts, 1.0) + b3_ref[...]
    m = jnp.max(pooled, axis=1, keepdims=True)
    lse = jnp.log(jnp.sum(jnp.exp(pooled - m), axis=1, keepdims=True)) + m
    o_ref[...] = pooled - lse


def _tc(body, out_shape, *args):
    return pl.pallas_call(body, out_shape=out_shape)(*args)


def kernel(x, edge_index, batch, W1, b1, g1, be1, W2, b2, g2, be2, W3, b3):
    f32 = jnp.float32
    ei3 = edge_index.reshape(2 * _NW, _NBAT, _K)
    ones_k = jnp.ones((_K, 16), f32)
    zeros_n = jnp.zeros((_NP, 16), f32)

    degp = _make_deg()(ei3, ones_k, zeros_n)
    zt1 = _tc(_mm_scale_body, jax.ShapeDtypeStruct((_NP, 64), f32),
              degp, x, W1)

    p1 = _make_agg(64)(ei3, zt1)
    zt2 = _tc(_bn_relu_scale_body, jax.ShapeDtypeStruct((_NP, 64), f32),
              degp, p1, zt1, g1.reshape(1, -1), be1.reshape(1, -1),
              b1.reshape(1, -1))

    p2 = _make_agg(64)(ei3, zt2)
    W3p = jnp.pad(W3, ((0, 0), (0, 16 - W3.shape[1])))
    zt3 = _tc(_layer2_body, jax.ShapeDtypeStruct((_NP, 16), f32),
              degp, p2, zt2, W2, b2.reshape(1, -1), g2.reshape(1, -1),
              be2.reshape(1, -1), W3p)

    p3 = _make_agg(16)(ei3, zt3)
    out = _tc(_pool_body, jax.ShapeDtypeStruct((64, 10), f32),
              degp, p3, zt3, batch.reshape(_N, 1), b3.reshape(1, -1))
    return out


# speedup vs baseline: 46.1737x; 1.0189x over previous
import functools

import jax
import jax.numpy as jnp
from jax import lax
from jax.experimental import pallas as pl
from jax.experimental.pallas import tpu as pltpu
from jax.experimental.pallas import tpu_sc as plsc

_N = 10000
_NP = 10240
_E = 320000
_NC = 2
_NS = 16
_NW = _NC * _NS
_EPW = _E // _NW
_K = 125
_NBAT = _EPW // _K
_RPT = _NP // _NS
_NBUF = 8


@functools.lru_cache(maxsize=None)
def _mesh():
    return plsc.VectorSubcoreMesh(
        core_axis_name="c", subcore_axis_name="s", num_cores=_NC, num_subcores=_NS
    )


@functools.lru_cache(maxsize=None)
def _make_agg(F):
    @functools.partial(
        pl.kernel,
        out_type=jax.ShapeDtypeStruct((_NC, _NP, F), jnp.float32),
        mesh=_mesh(),
        compiler_params=pltpu.CompilerParams(use_tc_tiling_on_sc=False),
        scratch_types=[
            pltpu.VMEM((_NBAT, _K), jnp.int32),
            pltpu.VMEM((_NBAT, _K), jnp.int32),
            [pltpu.VMEM((_K, F), jnp.float32)] * _NBUF,
            [pltpu.SemaphoreType.DMA] * _NBUF,
            [pltpu.SemaphoreType.DMA] * _NBUF,
            pltpu.VMEM_SHARED((_NP, F), jnp.float32),
        ],
    )
    def agg(ei_hbm, zt_hbm, out_hbm, src_v, dst_v, rows, gsem, ssem, acc):
        c = lax.axis_index("c")
        s = lax.axis_index("s")
        wid = s * _NC + c
        row0 = pl.multiple_of(s * _RPT, _RPT)
        pltpu.sync_copy(ei_hbm.at[wid], src_v)
        pltpu.sync_copy(ei_hbm.at[_NW + wid], dst_v)
        pltpu.sync_copy(zt_hbm.at[pl.ds(row0, _RPT)], acc.at[pl.ds(row0, _RPT)])
        plsc.subcore_barrier()

        for b in range(_NBUF):
            pltpu.async_copy(zt_hbm.at[src_v.at[b]], rows[b], gsem[b])

        def body(j, carry):
            for b in range(_NBUF):
                i = _NBUF * j + b
                pltpu.make_async_copy(zt_hbm.at[src_v.at[i]], rows[b],
                                      gsem[b]).wait()
                pltpu.async_copy(rows[b], acc.at[dst_v.at[i]], ssem[b],
                                 add=True)

                @pl.when(i + _NBUF < _NBAT)
                def _():
                    pltpu.make_async_copy(rows[b], acc.at[dst_v.at[i]],
                                          ssem[b]).wait()
                    pltpu.async_copy(zt_hbm.at[src_v.at[i + _NBUF]], rows[b],
                                     gsem[b])

            return carry

        lax.fori_loop(0, _NBAT // _NBUF, body, 0)
        for b in range(_NBUF):
            i = _NBAT - _NBUF + b
            pltpu.make_async_copy(rows[b], acc.at[dst_v.at[i]], ssem[b]).wait()
        plsc.subcore_barrier()
        pltpu.sync_copy(acc.at[pl.ds(row0, _RPT)], out_hbm.at[c, pl.ds(row0, _RPT)])

    return agg


@functools.lru_cache(maxsize=None)
def _make_deg():
    @functools.partial(
        pl.kernel,
        out_type=jax.ShapeDtypeStruct((_NC, _NP, 16), jnp.float32),
        mesh=_mesh(),
        compiler_params=pltpu.CompilerParams(use_tc_tiling_on_sc=False),
        scratch_types=[
            pltpu.VMEM((_NBAT, _K), jnp.int32),
            pltpu.VMEM((_K, 16), jnp.float32),
            pltpu.VMEM_SHARED((_NP, 16), jnp.float32),
            pltpu.SemaphoreType.DMA,
        ],
    )
    def deg_kernel(ei_hbm, ones_hbm, zeros_hbm, out_hbm, dst_v, ones_v, acc,
                   sem):
        c = lax.axis_index("c")
        s = lax.axis_index("s")
        wid = s * _NC + c
        row0 = pl.multiple_of(s * _RPT, _RPT)
        pltpu.sync_copy(ei_hbm.at[_NW + wid], dst_v)
        pltpu.sync_copy(ones_hbm, ones_v)
        pltpu.sync_copy(zeros_hbm.at[pl.ds(row0, _RPT)], acc.at[pl.ds(row0, _RPT)])
        plsc.subcore_barrier()

        def body(j, carry):
            for t in range(16):
                pltpu.async_copy(ones_v, acc.at[dst_v.at[16 * j + t]], sem,
                                 add=True)
            for t in range(16):
                pltpu.make_async_copy(ones_v, acc.at[dst_v.at[16 * j + t]],
                                      sem).wait()
            return carry

        lax.fori_loop(0, _NBAT // 16, body, 0)
        plsc.subcore_barrier()
        pltpu.sync_copy(acc.at[pl.ds(row0, _RPT)], out_hbm.at[c, pl.ds(row0, _RPT)])

    return deg_kernel


def _dis_of(degp_ref):
    deg = degp_ref[0, 0:_N, 0:1] + degp_ref[1, 0:_N, 0:1] + 1.0
    return lax.rsqrt(deg)


def _mm_scale_body(degp_ref, x_ref, w_ref, o_ref):
    y = jnp.dot(x_ref[...], w_ref[...], preferred_element_type=jnp.float32)
    o_ref[0:_N, :] = _dis_of(degp_ref) * y
    o_ref[_N:, :] = jnp.zeros((_NP - _N, o_ref.shape[1]), jnp.float32)


def _bn_relu_scale_body(degp_ref, p_ref, zt_ref, g_ref, be_ref, b_ref, o_ref):
    dis = _dis_of(degp_ref)
    agg = p_ref[0, 0:_N, :] + p_ref[1, 0:_N, :] - zt_ref[0:_N, :]
    t = dis * agg + b_ref[...]
    mu = jnp.mean(t, axis=0, keepdims=True)
    var = jnp.mean((t - mu) ** 2, axis=0, keepdims=True)
    h = (t - mu) * lax.rsqrt(var + 1e-5) * g_ref[...] + be_ref[...]
    o_ref[0:_N, :] = dis * jnp.maximum(h, 0.0)
    o_ref[_N:, :] = jnp.zeros((_NP - _N, o_ref.shape[1]), jnp.float32)


def _layer2_body(degp_ref, p_ref, zt_ref, w2_ref, b2_ref, g2_ref, be2_ref,
                 w3_ref, o_ref):
    dis = _dis_of(degp_ref)
    u = dis * (p_ref[0, 0:_N, :] + p_ref[1, 0:_N, :] - zt_ref[0:_N, :])
    t = jnp.dot(u, w2_ref[...], preferred_element_type=jnp.float32) + b2_ref[...]
    mu = jnp.mean(t, axis=0, keepdims=True)
    var = jnp.mean((t - mu) ** 2, axis=0, keepdims=True)
    h = (t - mu) * lax.rsqrt(var + 1e-5) * g2_ref[...] + be2_ref[...]
    h = jnp.maximum(h, 0.0)
    o_ref[0:_N, :] = dis * jnp.dot(h, w3_ref[...], preferred_element_type=jnp.float32)
    o_ref[_N:, :] = jnp.zeros((_NP - _N, o_ref.shape[1]), jnp.float32)


def _pool_body(degp_ref, p_ref, zt_ref, batch_ref, b3_ref, o_ref):
    dis = _dis_of(degp_ref)
    t3 = dis * (p_ref[0, 0:_N, :] + p_ref[1, 0:_N, :] - zt_ref[0:_N, :])
    gids = lax.broadcasted_iota(jnp.int32, (64, 1), 0)
    oht = (gids == batch_ref[...]).astype(jnp.float32)
    sums = jnp.dot(oht, t3, preferred_element_type=jnp.float32)
    counts = jnp.dot(oht, jnp.ones((_N, 1), jnp.float32),
                     preferred_element_type=jnp.float32)
    pooled = sums[:, :10] / jnp.maximum(counts, 1.0) + b3_ref[...]
    m = jnp.max(pooled, axis=1, keepdims=True)
    lse = jnp.log(jnp.sum(jnp.exp(pooled - m), axis=1, keepdims=True)) + m
    o_ref[...] = pooled - lse


def _tc(body, out_shape, *args):
    return pl.pallas_call(body, out_shape=out_shape)(*args)


def kernel(x, edge_index, batch, W1, b1, g1, be1, W2, b2, g2, be2, W3, b3):
    f32 = jnp.float32
    ei3 = edge_index.reshape(2 * _NW, _NBAT, _K)
    ones_k = jnp.ones((_K, 16), f32)
    zeros_n = jnp.zeros((_NP, 16), f32)

    degp = _make_deg()(ei3, ones_k, zeros_n)
    zt1 = _tc(_mm_scale_body, jax.ShapeDtypeStruct((_NP, 64), f32),
              degp, x, W1)

    p1 = _make_agg(64)(ei3, zt1)
    zt2 = _tc(_bn_relu_scale_body, jax.ShapeDtypeStruct((_NP, 64), f32),
              degp, p1, zt1, g1.reshape(1, -1), be1.reshape(1, -1),
              b1.reshape(1, -1))

    p2 = _make_agg(64)(ei3, zt2)
    W3p = jnp.pad(W3, ((0, 0), (0, 16 - W3.shape[1])))
    zt3 = _tc(_layer2_body, jax.ShapeDtypeStruct((_NP, 16), f32),
              degp, p2, zt2, W2, b2.reshape(1, -1), g2.reshape(1, -1),
              be2.reshape(1, -1), W3p)

    p3 = _make_agg(16)(ei3, zt3)
    out = _tc(_pool_body, jax.ShapeDtypeStruct((64, 10), f32),
              degp, p3, zt3, batch.reshape(1, _N), b3.reshape(1, -1))
    return out


# speedup vs baseline: 46.8886x; 1.0155x over previous
import functools

import jax
import jax.numpy as jnp
from jax import lax
from jax.experimental import pallas as pl
from jax.experimental.pallas import tpu as pltpu
from jax.experimental.pallas import tpu_sc as plsc

_N = 10000
_NP = 10240
_E = 320000
_NC = 2
_NS = 16
_NW = _NC * _NS
_EPW = _E // _NW
_K = 125
_NBAT = _EPW // _K
_RPT = _NP // _NS
_NBUF = 8


@functools.lru_cache(maxsize=None)
def _mesh():
    return plsc.VectorSubcoreMesh(
        core_axis_name="c", subcore_axis_name="s", num_cores=_NC, num_subcores=_NS
    )


@functools.lru_cache(maxsize=None)
def _make_agg(F):
    @functools.partial(
        pl.kernel,
        out_type=jax.ShapeDtypeStruct((_NC, _NP, F), jnp.float32),
        mesh=_mesh(),
        compiler_params=pltpu.CompilerParams(use_tc_tiling_on_sc=False),
        scratch_types=[
            pltpu.VMEM((_NBAT, _K), jnp.int32),
            pltpu.VMEM((_NBAT, _K), jnp.int32),
            [pltpu.VMEM((_K, F), jnp.float32)] * _NBUF,
            [pltpu.SemaphoreType.DMA] * _NBUF,
            [pltpu.SemaphoreType.DMA] * _NBUF,
            pltpu.VMEM_SHARED((_NP, F), jnp.float32),
        ],
    )
    def agg(ei_hbm, zt_hbm, zeros_hbm, out_hbm, src_v, dst_v, rows, gsem,
            ssem, acc):
        c = lax.axis_index("c")
        s = lax.axis_index("s")
        wid = s * _NC + c
        row0 = pl.multiple_of(s * _RPT, _RPT)
        pltpu.sync_copy(ei_hbm.at[wid], src_v)
        pltpu.sync_copy(ei_hbm.at[_NW + wid], dst_v)

        @pl.when(c == 0)
        def _():
            pltpu.sync_copy(zt_hbm.at[pl.ds(row0, _RPT)],
                            acc.at[pl.ds(row0, _RPT)])

        @pl.when(c == 1)
        def _():
            pltpu.sync_copy(zeros_hbm.at[pl.ds(row0, _RPT)],
                            acc.at[pl.ds(row0, _RPT)])

        plsc.subcore_barrier()

        for b in range(_NBUF):
            pltpu.async_copy(zt_hbm.at[src_v.at[b]], rows[b], gsem[b])

        def body(j, carry):
            for b in range(_NBUF):
                i = _NBUF * j + b
                pltpu.make_async_copy(zt_hbm.at[src_v.at[i]], rows[b],
                                      gsem[b]).wait()
                pltpu.async_copy(rows[b], acc.at[dst_v.at[i]], ssem[b],
                                 add=True)

                @pl.when(i + _NBUF < _NBAT)
                def _():
                    pltpu.make_async_copy(rows[b], acc.at[dst_v.at[i]],
                                          ssem[b]).wait()
                    pltpu.async_copy(zt_hbm.at[src_v.at[i + _NBUF]], rows[b],
                                     gsem[b])

            return carry

        lax.fori_loop(0, _NBAT // _NBUF, body, 0)
        for b in range(_NBUF):
            i = _NBAT - _NBUF + b
            pltpu.make_async_copy(rows[b], acc.at[dst_v.at[i]], ssem[b]).wait()
        plsc.subcore_barrier()
        pltpu.sync_copy(acc.at[pl.ds(row0, _RPT)], out_hbm.at[c, pl.ds(row0, _RPT)])

    return agg


@functools.lru_cache(maxsize=None)
def _make_deg():
    @functools.partial(
        pl.kernel,
        out_type=jax.ShapeDtypeStruct((_NC, _NP, 16), jnp.float32),
        mesh=_mesh(),
        compiler_params=pltpu.CompilerParams(use_tc_tiling_on_sc=False),
        scratch_types=[
            pltpu.VMEM((_NBAT, _K), jnp.int32),
            pltpu.VMEM((_K, 16), jnp.float32),
            pltpu.VMEM_SHARED((_NP, 16), jnp.float32),
            pltpu.SemaphoreType.DMA,
        ],
    )
    def deg_kernel(ei_hbm, ones_hbm, zeros_hbm, out_hbm, dst_v, ones_v, acc,
                   sem):
        c = lax.axis_index("c")
        s = lax.axis_index("s")
        wid = s * _NC + c
        row0 = pl.multiple_of(s * _RPT, _RPT)
        pltpu.sync_copy(ei_hbm.at[_NW + wid], dst_v)
        pltpu.sync_copy(ones_hbm, ones_v)
        pltpu.sync_copy(zeros_hbm.at[pl.ds(row0, _RPT)], acc.at[pl.ds(row0, _RPT)])
        plsc.subcore_barrier()

        def body(j, carry):
            for t in range(16):
                pltpu.async_copy(ones_v, acc.at[dst_v.at[16 * j + t]], sem,
                                 add=True)
            for t in range(16):
                pltpu.make_async_copy(ones_v, acc.at[dst_v.at[16 * j + t]],
                                      sem).wait()
            return carry

        lax.fori_loop(0, _NBAT // 16, body, 0)
        plsc.subcore_barrier()
        pltpu.sync_copy(acc.at[pl.ds(row0, _RPT)], out_hbm.at[c, pl.ds(row0, _RPT)])

    return deg_kernel


def _mm_scale_body(degp_ref, x_ref, w_ref, o_ref, dis_ref):
    deg = degp_ref[0, :, 0:1] + degp_ref[1, :, 0:1] + 1.0
    dis = lax.rsqrt(deg)
    dis_ref[...] = jnp.broadcast_to(dis, (_NP, 64))
    y = jnp.dot(x_ref[...], w_ref[...], preferred_element_type=jnp.float32)
    o_ref[0:_N, :] = dis[0:_N] * y
    o_ref[_N:, :] = jnp.zeros((_NP - _N, o_ref.shape[1]), jnp.float32)


def _bn_relu_scale_body(dis_ref, p_ref, g_ref, be_ref, b_ref, o_ref):
    dis = dis_ref[0:_N, :]
    t = dis * (p_ref[0, 0:_N, :] + p_ref[1, 0:_N, :]) + b_ref[...]
    mu = jnp.mean(t, axis=0, keepdims=True)
    var = jnp.mean((t - mu) ** 2, axis=0, keepdims=True)
    h = (t - mu) * lax.rsqrt(var + 1e-5) * g_ref[...] + be_ref[...]
    o_ref[0:_N, :] = dis * jnp.maximum(h, 0.0)
    o_ref[_N:, :] = jnp.zeros((_NP - _N, o_ref.shape[1]), jnp.float32)


def _layer2_body(dis_ref, p_ref, w2_ref, b2_ref, g2_ref, be2_ref,
                 w3_ref, o_ref):
    dis = dis_ref[0:_N, :]
    u = dis * (p_ref[0, 0:_N, :] + p_ref[1, 0:_N, :])
    t = jnp.dot(u, w2_ref[...], preferred_element_type=jnp.float32) + b2_ref[...]
    mu = jnp.mean(t, axis=0, keepdims=True)
    var = jnp.mean((t - mu) ** 2, axis=0, keepdims=True)
    h = (t - mu) * lax.rsqrt(var + 1e-5) * g2_ref[...] + be2_ref[...]
    h = jnp.maximum(h, 0.0)
    o_ref[0:_N, :] = dis[:, 0:16] * jnp.dot(
        h, w3_ref[...], preferred_element_type=jnp.float32)
    o_ref[_N:, :] = jnp.zeros((_NP - _N, o_ref.shape[1]), jnp.float32)


def _pool_body(dis_ref, p_ref, batch_ref, b3_ref, o_ref):
    t3 = dis_ref[0:_N, 0:16] * (p_ref[0, 0:_N, :] + p_ref[1, 0:_N, :])
    gids = lax.broadcasted_iota(jnp.int32, (64, 1), 0)
    oht = (gids == batch_ref[...]).astype(jnp.float32)
    sums = jnp.dot(oht, t3, preferred_element_type=jnp.float32)
    counts = jnp.dot(oht, jnp.ones((_N, 1), jnp.float32),
                     preferred_element_type=jnp.float32)
    pooled = sums[:, :10] / jnp.maximum(counts, 1.0) + b3_ref[...]
    m = jnp.max(pooled, axis=1, keepdims=True)
    lse = jnp.log(jnp.sum(jnp.exp(pooled - m), axis=1, keepdims=True)) + m
    o_ref[...] = pooled - lse


def _tc(body, out_shape, *args):
    return pl.pallas_call(body, out_shape=out_shape)(*args)


def kernel(x, edge_index, batch, W1, b1, g1, be1, W2, b2, g2, be2, W3, b3):
    f32 = jnp.float32
    ei3 = edge_index.reshape(2 * _NW, _NBAT, _K)
    ones_k = jnp.ones((_K, 16), f32)
    zeros_n16 = jnp.zeros((_NP, 16), f32)
    zeros_n64 = jnp.zeros((_NP, 64), f32)

    degp = _make_deg()(ei3, ones_k, zeros_n16)
    zt1, dis64 = _tc(_mm_scale_body,
                     (jax.ShapeDtypeStruct((_NP, 64), f32),
                      jax.ShapeDtypeStruct((_NP, 64), f32)),
                     degp, x, W1)

    p1 = _make_agg(64)(ei3, zt1, zeros_n64)
    zt2 = _tc(_bn_relu_scale_body, jax.ShapeDtypeStruct((_NP, 64), f32),
              dis64, p1, g1.reshape(1, -1), be1.reshape(1, -1),
              b1.reshape(1, -1))

    p2 = _make_agg(64)(ei3, zt2, zeros_n64)
    W3p = jnp.pad(W3, ((0, 0), (0, 16 - W3.shape[1])))
    zt3 = _tc(_layer2_body, jax.ShapeDtypeStruct((_NP, 16), f32),
              dis64, p2, W2, b2.reshape(1, -1), g2.reshape(1, -1),
              be2.reshape(1, -1), W3p)

    p3 = _make_agg(16)(ei3, zt3, zeros_n16)
    out = _tc(_pool_body, jax.ShapeDtypeStruct((64, 10), f32),
              dis64, p3, batch.reshape(1, _N), b3.reshape(1, -1))
    return out


# speedup vs baseline: 47.0330x; 1.0031x over previous
import functools

import jax
import jax.numpy as jnp
from jax import lax
from jax.experimental import pallas as pl
from jax.experimental.pallas import tpu as pltpu
from jax.experimental.pallas import tpu_sc as plsc

_N = 10000
_NP = 10240
_E = 320000
_NC = 2
_NS = 16
_NW = _NC * _NS
_EPW = _E // _NW
_K = 125
_NBAT = _EPW // _K
_RPT = _NP // _NS
_NBUF = 8


@functools.lru_cache(maxsize=None)
def _mesh():
    return plsc.VectorSubcoreMesh(
        core_axis_name="c", subcore_axis_name="s", num_cores=_NC, num_subcores=_NS
    )


@functools.lru_cache(maxsize=None)
def _make_agg(F):
    @functools.partial(
        pl.kernel,
        out_type=jax.ShapeDtypeStruct((_NC, _NP, F), jnp.float32),
        mesh=_mesh(),
        compiler_params=pltpu.CompilerParams(use_tc_tiling_on_sc=False),
        scratch_types=[
            pltpu.VMEM((_NBAT, _K), jnp.int32),
            pltpu.VMEM((_NBAT, _K), jnp.int32),
            [pltpu.VMEM((_K, F), jnp.float32)] * _NBUF,
            [pltpu.SemaphoreType.DMA] * _NBUF,
            [pltpu.SemaphoreType.DMA] * _NBUF,
            pltpu.VMEM_SHARED((_NP, F), jnp.float32),
        ],
    )
    def agg(ei_hbm, zt_hbm, zeros_hbm, out_hbm, src_v, dst_v, rows, gsem,
            ssem, acc):
        c = lax.axis_index("c")
        s = lax.axis_index("s")
        wid = s * _NC + c
        row0 = pl.multiple_of(s * _RPT, _RPT)
        pltpu.sync_copy(ei_hbm.at[wid], src_v)
        pltpu.sync_copy(ei_hbm.at[_NW + wid], dst_v)

        @pl.when(c == 0)
        def _():
            pltpu.sync_copy(zt_hbm.at[pl.ds(row0, _RPT)],
                            acc.at[pl.ds(row0, _RPT)])

        @pl.when(c == 1)
        def _():
            pltpu.sync_copy(zeros_hbm.at[pl.ds(row0, _RPT)],
                            acc.at[pl.ds(row0, _RPT)])

        plsc.subcore_barrier()

        for b in range(_NBUF):
            pltpu.async_copy(zt_hbm.at[src_v.at[b]], rows[b], gsem[b])

        def body(j, carry):
            for b in range(_NBUF):
                i = _NBUF * j + b
                pltpu.make_async_copy(zt_hbm.at[src_v.at[i]], rows[b],
                                      gsem[b]).wait()
                pltpu.async_copy(rows[b], acc.at[dst_v.at[i]], ssem[b],
                                 add=True)

                @pl.when(i + _NBUF < _NBAT)
                def _():
                    pltpu.make_async_copy(rows[b], acc.at[dst_v.at[i]],
                                          ssem[b]).wait()
                    pltpu.async_copy(zt_hbm.at[src_v.at[i + _NBUF]], rows[b],
                                     gsem[b])

            return carry

        lax.fori_loop(0, _NBAT // _NBUF, body, 0)
        for b in range(_NBUF):
            i = _NBAT - _NBUF + b
            pltpu.make_async_copy(rows[b], acc.at[dst_v.at[i]], ssem[b]).wait()
        plsc.subcore_barrier()
        pltpu.sync_copy(acc.at[pl.ds(row0, _RPT)], out_hbm.at[c, pl.ds(row0, _RPT)])

    return agg


@functools.lru_cache(maxsize=None)
def _make_deg():
    @functools.partial(
        pl.kernel,
        out_type=jax.ShapeDtypeStruct((_NC, _NP, 16), jnp.float32),
        mesh=_mesh(),
        compiler_params=pltpu.CompilerParams(use_tc_tiling_on_sc=False),
        scratch_types=[
            pltpu.VMEM((_NBAT, _K), jnp.int32),
            pltpu.VMEM((_K, 16), jnp.float32),
            pltpu.VMEM_SHARED((_NP, 16), jnp.float32),
            pltpu.SemaphoreType.DMA,
        ],
    )
    def deg_kernel(ei_hbm, ones_hbm, zeros_hbm, out_hbm, dst_v, ones_v, acc,
                   sem):
        c = lax.axis_index("c")
        s = lax.axis_index("s")
        wid = s * _NC + c
        row0 = pl.multiple_of(s * _RPT, _RPT)
        pltpu.sync_copy(ei_hbm.at[_NW + wid], dst_v)
        pltpu.sync_copy(ones_hbm, ones_v)
        pltpu.sync_copy(zeros_hbm.at[pl.ds(row0, _RPT)], acc.at[pl.ds(row0, _RPT)])
        plsc.subcore_barrier()

        def body(j, carry):
            for t in range(16):
                pltpu.async_copy(ones_v, acc.at[dst_v.at[16 * j + t]], sem,
                                 add=True)
            for t in range(16):
                pltpu.make_async_copy(ones_v, acc.at[dst_v.at[16 * j + t]],
                                      sem).wait()
            return carry

        lax.fori_loop(0, _NBAT // 16, body, 0)
        plsc.subcore_barrier()
        pltpu.sync_copy(acc.at[pl.ds(row0, _RPT)], out_hbm.at[c, pl.ds(row0, _RPT)])

    return deg_kernel


def _mm_body(x_ref, w_ref, o_ref):
    o_ref[...] = jnp.dot(x_ref[...], w_ref[...], preferred_element_type=jnp.float32)


def _scale_body(degp_ref, y_ref, o_ref, dis_ref):
    deg = degp_ref[0, :, 0:1] + degp_ref[1, :, 0:1] + 1.0
    dis = lax.rsqrt(deg)
    dis_ref[...] = jnp.broadcast_to(dis, (_NP, 64))
    o_ref[0:_N, :] = dis[0:_N] * y_ref[...]
    o_ref[_N:, :] = jnp.zeros((_NP - _N, o_ref.shape[1]), jnp.float32)


def _bn_relu_scale_body(dis_ref, p_ref, g_ref, be_ref, b_ref, o_ref):
    dis = dis_ref[0:_N, :]
    t = dis * (p_ref[0, 0:_N, :] + p_ref[1, 0:_N, :]) + b_ref[...]
    mu = jnp.mean(t, axis=0, keepdims=True)
    var = jnp.mean((t - mu) ** 2, axis=0, keepdims=True)
    h = (t - mu) * lax.rsqrt(var + 1e-5) * g_ref[...] + be_ref[...]
    o_ref[0:_N, :] = dis * jnp.maximum(h, 0.0)
    o_ref[_N:, :] = jnp.zeros((_NP - _N, o_ref.shape[1]), jnp.float32)


def _layer2_body(dis_ref, p_ref, w2_ref, b2_ref, g2_ref, be2_ref,
                 w3_ref, o_ref):
    dis = dis_ref[0:_N, :]
    u = dis * (p_ref[0, 0:_N, :] + p_ref[1, 0:_N, :])
    t = jnp.dot(u, w2_ref[...], preferred_element_type=jnp.float32) + b2_ref[...]
    mu = jnp.mean(t, axis=0, keepdims=True)
    var = jnp.mean((t - mu) ** 2, axis=0, keepdims=True)
    h = (t - mu) * lax.rsqrt(var + 1e-5) * g2_ref[...] + be2_ref[...]
    h = jnp.maximum(h, 0.0)
    o_ref[0:_N, :] = dis[:, 0:16] * jnp.dot(
        h, w3_ref[...], preferred_element_type=jnp.float32)
    o_ref[_N:, :] = jnp.zeros((_NP - _N, o_ref.shape[1]), jnp.float32)


def _pool_body(dis_ref, p_ref, batch_ref, b3_ref, o_ref):
    t3 = dis_ref[0:_N, 0:16] * (p_ref[0, 0:_N, :] + p_ref[1, 0:_N, :])
    gids = lax.broadcasted_iota(jnp.int32, (64, 1), 0)
    oht = (gids == batch_ref[...]).astype(jnp.float32)
    sums = jnp.dot(oht, t3, preferred_element_type=jnp.float32)
    counts = jnp.dot(oht, jnp.ones((_N, 1), jnp.float32),
                     preferred_element_type=jnp.float32)
    pooled = sums[:, :10] / jnp.maximum(counts, 1.0) + b3_ref[...]
    m = jnp.max(pooled, axis=1, keepdims=True)
    lse = jnp.log(jnp.sum(jnp.exp(pooled - m), axis=1, keepdims=True)) + m
    o_ref[...] = pooled - lse


def _tc(body, out_shape, *args):
    return pl.pallas_call(body, out_shape=out_shape)(*args)


def kernel(x, edge_index, batch, W1, b1, g1, be1, W2, b2, g2, be2, W3, b3):
    f32 = jnp.float32
    ei3 = edge_index.reshape(2 * _NW, _NBAT, _K)
    ones_k = jnp.ones((_K, 16), f32)
    zeros_n16 = jnp.zeros((_NP, 16), f32)
    zeros_n64 = jnp.zeros((_NP, 64), f32)

    degp = _make_deg()(ei3, ones_k, zeros_n16)
    y1 = _tc(_mm_body, jax.ShapeDtypeStruct((_N, 64), f32), x, W1)
    zt1, dis64 = _tc(_scale_body,
                     (jax.ShapeDtypeStruct((_NP, 64), f32),
                      jax.ShapeDtypeStruct((_NP, 64), f32)),
                     degp, y1)

    p1 = _make_agg(64)(ei3, zt1, zeros_n64)
    zt2 = _tc(_bn_relu_scale_body, jax.ShapeDtypeStruct((_NP, 64), f32),
              dis64, p1, g1.reshape(1, -1), be1.reshape(1, -1),
              b1.reshape(1, -1))

    p2 = _make_agg(64)(ei3, zt2, zeros_n64)
    W3p = jnp.pad(W3, ((0, 0), (0, 16 - W3.shape[1])))
    zt3 = _tc(_layer2_body, jax.ShapeDtypeStruct((_NP, 16), f32),
              dis64, p2, W2, b2.reshape(1, -1), g2.reshape(1, -1),
              be2.reshape(1, -1), W3p)

    p3 = _make_agg(16)(ei3, zt3, zeros_n16)
    out = _tc(_pool_body, jax.ShapeDtypeStruct((64, 10), f32),
              dis64, p3, batch.reshape(1, _N), b3.reshape(1, -1))
    return out


# speedup vs baseline: 47.0609x; 1.0006x over previous
import functools

import jax
import jax.numpy as jnp
from jax import lax
from jax.experimental import pallas as pl
from jax.experimental.pallas import tpu as pltpu
from jax.experimental.pallas import tpu_sc as plsc

_N = 10000
_NP = 10240
_E = 320000
_NC = 2
_NS = 16
_NW = _NC * _NS
_EPW = _E // _NW
_K = 125
_NBAT = _EPW // _K
_RPT = _NP // _NS
_NBUF = 8


@functools.lru_cache(maxsize=None)
def _mesh():
    return plsc.VectorSubcoreMesh(
        core_axis_name="c", subcore_axis_name="s", num_cores=_NC, num_subcores=_NS
    )


@functools.lru_cache(maxsize=None)
def _make_agg(F):
    @functools.partial(
        pl.kernel,
        out_type=jax.ShapeDtypeStruct((_NC, _NP, F), jnp.float32),
        mesh=_mesh(),
        compiler_params=pltpu.CompilerParams(use_tc_tiling_on_sc=False),
        scratch_types=[
            pltpu.VMEM((_NBAT, _K), jnp.int32),
            pltpu.VMEM((_NBAT, _K), jnp.int32),
            [pltpu.VMEM((_K, F), jnp.float32)] * _NBUF,
            [pltpu.SemaphoreType.DMA] * _NBUF,
            [pltpu.SemaphoreType.DMA] * _NBUF,
            pltpu.VMEM_SHARED((_NP, F), jnp.float32),
        ],
    )
    def agg(ei_hbm, zt_hbm, zeros_hbm, out_hbm, src_v, dst_v, rows, gsem,
            ssem, acc):
        c = lax.axis_index("c")
        s = lax.axis_index("s")
        wid = s * _NC + c
        row0 = pl.multiple_of(s * _RPT, _RPT)
        pltpu.sync_copy(ei_hbm.at[wid], src_v)
        pltpu.sync_copy(ei_hbm.at[_NW + wid], dst_v)

        @pl.when(c == 0)
        def _():
            pltpu.sync_copy(zt_hbm.at[pl.ds(row0, _RPT)],
                            acc.at[pl.ds(row0, _RPT)])

        @pl.when(c == 1)
        def _():
            pltpu.sync_copy(zeros_hbm.at[pl.ds(row0, _RPT)],
                            acc.at[pl.ds(row0, _RPT)])

        plsc.subcore_barrier()

        for b in range(_NBUF):
            pltpu.async_copy(zt_hbm.at[src_v.at[b]], rows[b], gsem[b])

        def body(j, carry):
            for b in range(_NBUF):
                i = _NBUF * j + b
                pltpu.make_async_copy(zt_hbm.at[src_v.at[i]], rows[b],
                                      gsem[b]).wait()
                pltpu.async_copy(rows[b], acc.at[dst_v.at[i]], ssem[b],
                                 add=True)

                @pl.when(i + _NBUF < _NBAT)
                def _():
                    pltpu.make_async_copy(rows[b], acc.at[dst_v.at[i]],
                                          ssem[b]).wait()
                    pltpu.async_copy(zt_hbm.at[src_v.at[i + _NBUF]], rows[b],
                                     gsem[b])

            return carry

        lax.fori_loop(0, _NBAT // _NBUF, body, 0)
        for b in range(_NBUF):
            i = _NBAT - _NBUF + b
            pltpu.make_async_copy(rows[b], acc.at[dst_v.at[i]], ssem[b]).wait()
        plsc.subcore_barrier()
        pltpu.sync_copy(acc.at[pl.ds(row0, _RPT)], out_hbm.at[c, pl.ds(row0, _RPT)])

    return agg


@functools.lru_cache(maxsize=None)
def _make_deg():
    @functools.partial(
        pl.kernel,
        out_type=jax.ShapeDtypeStruct((_NC, _NP, 16), jnp.float32),
        mesh=_mesh(),
        compiler_params=pltpu.CompilerParams(use_tc_tiling_on_sc=False),
        scratch_types=[
            pltpu.VMEM((_NBAT, _K), jnp.int32),
            pltpu.VMEM((_K, 16), jnp.float32),
            pltpu.VMEM_SHARED((_NP, 16), jnp.float32),
            pltpu.SemaphoreType.DMA,
        ],
    )
    def deg_kernel(ei_hbm, ones_hbm, zeros_hbm, out_hbm, dst_v, ones_v, acc,
                   sem):
        c = lax.axis_index("c")
        s = lax.axis_index("s")
        wid = s * _NC + c
        row0 = pl.multiple_of(s * _RPT, _RPT)
        pltpu.sync_copy(ei_hbm.at[_NW + wid], dst_v)
        pltpu.sync_copy(ones_hbm, ones_v)
        pltpu.sync_copy(zeros_hbm.at[pl.ds(row0, _RPT)], acc.at[pl.ds(row0, _RPT)])
        plsc.subcore_barrier()

        def body(j, carry):
            for t in range(16):
                pltpu.async_copy(ones_v, acc.at[dst_v.at[16 * j + t]], sem,
                                 add=True)
            for t in range(16):
                pltpu.make_async_copy(ones_v, acc.at[dst_v.at[16 * j + t]],
                                      sem).wait()
            return carry

        lax.fori_loop(0, _NBAT // 16, body, 0)
        plsc.subcore_barrier()
        pltpu.sync_copy(acc.at[pl.ds(row0, _RPT)], out_hbm.at[c, pl.ds(row0, _RPT)])

    return deg_kernel


def _mm_body(x_ref, w_ref, o_ref):
    o_ref[...] = jnp.dot(x_ref[...], w_ref[...], preferred_element_type=jnp.float32)


def _scale_body(degp_ref, y_ref, o_ref, dis_ref):
    deg = degp_ref[0, :, 0:1] + degp_ref[1, :, 0:1] + 1.0
    dis = lax.rsqrt(deg)
    dis_ref[...] = jnp.broadcast_to(dis, (_NP, 64))
    o_ref[0:_N, :] = dis[0:_N] * y_ref[...]
    o_ref[_N:, :] = jnp.zeros((_NP - _N, o_ref.shape[1]), jnp.float32)


def _bn_relu_scale_body(dis_ref, p_ref, g_ref, be_ref, b_ref, o_ref):
    dis = dis_ref[0:_N, :]
    t = dis * (p_ref[0, 0:_N, :] + p_ref[1, 0:_N, :]) + b_ref[...]
    mu = jnp.mean(t, axis=0, keepdims=True)
    var = jnp.mean((t - mu) ** 2, axis=0, keepdims=True)
    h = (t - mu) * lax.rsqrt(var + 1e-5) * g_ref[...] + be_ref[...]
    o_ref[0:_N, :] = dis * jnp.maximum(h, 0.0)
    o_ref[_N:, :] = jnp.zeros((_NP - _N, o_ref.shape[1]), jnp.float32)


def _layer2_body(dis_ref, p_ref, w2_ref, b2_ref, g2_ref, be2_ref,
                 w3_ref, o_ref):
    dis = dis_ref[0:_N, :]
    u = dis * (p_ref[0, 0:_N, :] + p_ref[1, 0:_N, :])
    t = jnp.dot(u, w2_ref[...], preferred_element_type=jnp.float32) + b2_ref[...]
    mu = jnp.mean(t, axis=0, keepdims=True)
    var = jnp.mean((t - mu) ** 2, axis=0, keepdims=True)
    h = (t - mu) * lax.rsqrt(var + 1e-5) * g2_ref[...] + be2_ref[...]
    h = jnp.maximum(h, 0.0)
    o_ref[0:_N, 0:10] = dis[:, 0:10] * jnp.dot(
        h, w3_ref[...], preferred_element_type=jnp.float32)
    o_ref[0:_N, 10:16] = jnp.zeros((_N, 6), jnp.float32)
    o_ref[_N:, :] = jnp.zeros((_NP - _N, o_ref.shape[1]), jnp.float32)


def _pool_body(dis_ref, p_ref, batch_ref, b3_ref, o_ref):
    t3 = dis_ref[0:_N, 0:16] * (p_ref[0, 0:_N, :] + p_ref[1, 0:_N, :])
    gids = lax.broadcasted_iota(jnp.int32, (64, 1), 0)
    oht = (gids == batch_ref[...]).astype(jnp.float32)
    sums = jnp.dot(oht, t3, preferred_element_type=jnp.float32)
    counts = jnp.dot(oht, jnp.ones((_N, 1), jnp.float32),
                     preferred_element_type=jnp.float32)
    pooled = sums[:, :10] / jnp.maximum(counts, 1.0) + b3_ref[...]
    m = jnp.max(pooled, axis=1, keepdims=True)
    lse = jnp.log(jnp.sum(jnp.exp(pooled - m), axis=1, keepdims=True)) + m
    o_ref[...] = pooled - lse


def _tc(body, out_shape, *args):
    return pl.pallas_call(body, out_shape=out_shape)(*args)


def kernel(x, edge_index, batch, W1, b1, g1, be1, W2, b2, g2, be2, W3, b3):
    f32 = jnp.float32
    ei3 = edge_index.reshape(2 * _NW, _NBAT, _K)
    ones_k = jnp.ones((_K, 16), f32)
    zeros_n16 = jnp.zeros((_NP, 16), f32)
    zeros_n64 = jnp.zeros((_NP, 64), f32)

    degp = _make_deg()(ei3, ones_k, zeros_n16)
    y1 = _tc(_mm_body, jax.ShapeDtypeStruct((_N, 64), f32), x, W1)
    zt1, dis64 = _tc(_scale_body,
                     (jax.ShapeDtypeStruct((_NP, 64), f32),
                      jax.ShapeDtypeStruct((_NP, 64), f32)),
                     degp, y1)

    p1 = _make_agg(64)(ei3, zt1, zeros_n64)
    zt2 = _tc(_bn_relu_scale_body, jax.ShapeDtypeStruct((_NP, 64), f32),
              dis64, p1, g1.reshape(1, -1), be1.reshape(1, -1),
              b1.reshape(1, -1))

    p2 = _make_agg(64)(ei3, zt2, zeros_n64)
    zt3 = _tc(_layer2_body, jax.ShapeDtypeStruct((_NP, 16), f32),
              dis64, p2, W2, b2.reshape(1, -1), g2.reshape(1, -1),
              be2.reshape(1, -1), W3)

    p3 = _make_agg(16)(ei3, zt3, zeros_n16)
    out = _tc(_pool_body, jax.ShapeDtypeStruct((64, 10), f32),
              dis64, p3, batch.reshape(1, _N), b3.reshape(1, -1))
    return out


# speedup vs baseline: 47.0806x; 1.0004x over previous
import functools

import jax
import jax.numpy as jnp
from jax import lax
from jax.experimental import pallas as pl
from jax.experimental.pallas import tpu as pltpu
from jax.experimental.pallas import tpu_sc as plsc

_N = 10000
_NP = 10240
_E = 320000
_NC = 2
_NS = 16
_NW = _NC * _NS
_EPW = _E // _NW
_K = 125
_NBAT = _EPW // _K
_RPT = _NP // _NS
_NBUF = 8


@functools.lru_cache(maxsize=None)
def _mesh():
    return plsc.VectorSubcoreMesh(
        core_axis_name="c", subcore_axis_name="s", num_cores=_NC, num_subcores=_NS
    )


@functools.lru_cache(maxsize=None)
def _make_agg(F):
    nbuf = _NBUF

    @functools.partial(
        pl.kernel,
        out_type=jax.ShapeDtypeStruct((_NC, _NP, F), jnp.float32),
        mesh=_mesh(),
        compiler_params=pltpu.CompilerParams(use_tc_tiling_on_sc=False),
        scratch_types=[
            pltpu.VMEM((_NBAT, _K), jnp.int32),
            pltpu.VMEM((_NBAT, _K), jnp.int32),
            [pltpu.VMEM((_K, F), jnp.float32)] * nbuf,
            [pltpu.SemaphoreType.DMA] * nbuf,
            [pltpu.SemaphoreType.DMA] * nbuf,
            pltpu.VMEM_SHARED((_NP, F), jnp.float32),
        ],
    )
    def agg(ei_hbm, zt_hbm, zeros_hbm, out_hbm, src_v, dst_v, rows, gsem,
            ssem, acc):
        c = lax.axis_index("c")
        s = lax.axis_index("s")
        wid = s * _NC + c
        row0 = pl.multiple_of(s * _RPT, _RPT)
        pltpu.sync_copy(ei_hbm.at[wid], src_v)
        pltpu.sync_copy(ei_hbm.at[_NW + wid], dst_v)

        @pl.when(c == 0)
        def _():
            pltpu.sync_copy(zt_hbm.at[pl.ds(row0, _RPT)],
                            acc.at[pl.ds(row0, _RPT)])

        @pl.when(c == 1)
        def _():
            pltpu.sync_copy(zeros_hbm.at[pl.ds(row0, _RPT)],
                            acc.at[pl.ds(row0, _RPT)])

        plsc.subcore_barrier()

        for b in range(nbuf):
            pltpu.async_copy(zt_hbm.at[src_v.at[b]], rows[b], gsem[b])

        def body(j, carry):
            for b in range(nbuf):
                i = nbuf * j + b
                pltpu.make_async_copy(zt_hbm.at[src_v.at[i]], rows[b],
                                      gsem[b]).wait()
                pltpu.async_copy(rows[b], acc.at[dst_v.at[i]], ssem[b],
                                 add=True)

                @pl.when(i + nbuf < _NBAT)
                def _():
                    pltpu.make_async_copy(rows[b], acc.at[dst_v.at[i]],
                                          ssem[b]).wait()
                    pltpu.async_copy(zt_hbm.at[src_v.at[i + nbuf]], rows[b],
                                     gsem[b])

            return carry

        lax.fori_loop(0, _NBAT // nbuf, body, 0)
        for b in range(nbuf):
            i = _NBAT - nbuf + b
            pltpu.make_async_copy(rows[b], acc.at[dst_v.at[i]], ssem[b]).wait()
        plsc.subcore_barrier()
        pltpu.sync_copy(acc.at[pl.ds(row0, _RPT)], out_hbm.at[c, pl.ds(row0, _RPT)])

    return agg


@functools.lru_cache(maxsize=None)
def _make_deg():
    @functools.partial(
        pl.kernel,
        out_type=jax.ShapeDtypeStruct((_NC, _NP, 16), jnp.float32),
        mesh=_mesh(),
        compiler_params=pltpu.CompilerParams(use_tc_tiling_on_sc=False),
        scratch_types=[
            pltpu.VMEM((_NBAT, _K), jnp.int32),
            pltpu.VMEM((_K, 16), jnp.float32),
            pltpu.VMEM_SHARED((_NP, 16), jnp.float32),
            pltpu.SemaphoreType.DMA,
        ],
    )
    def deg_kernel(ei_hbm, ones_hbm, zeros_hbm, out_hbm, dst_v, ones_v, acc,
                   sem):
        c = lax.axis_index("c")
        s = lax.axis_index("s")
        wid = s * _NC + c
        row0 = pl.multiple_of(s * _RPT, _RPT)
        pltpu.sync_copy(ei_hbm.at[_NW + wid], dst_v)
        pltpu.sync_copy(ones_hbm, ones_v)
        pltpu.sync_copy(zeros_hbm.at[pl.ds(row0, _RPT)], acc.at[pl.ds(row0, _RPT)])
        plsc.subcore_barrier()

        def body(j, carry):
            for t in range(16):
                pltpu.async_copy(ones_v, acc.at[dst_v.at[16 * j + t]], sem,
                                 add=True)
            for t in range(16):
                pltpu.make_async_copy(ones_v, acc.at[dst_v.at[16 * j + t]],
                                      sem).wait()
            return carry

        lax.fori_loop(0, _NBAT // 16, body, 0)
        plsc.subcore_barrier()
        pltpu.sync_copy(acc.at[pl.ds(row0, _RPT)], out_hbm.at[c, pl.ds(row0, _RPT)])

    return deg_kernel


def _mm_body(x_ref, w_ref, o_ref):
    o_ref[...] = jnp.dot(x_ref[...], w_ref[...], preferred_element_type=jnp.float32)


def _scale_body(degp_ref, y_ref, o_ref, dis_ref):
    deg = degp_ref[0, :, 0:1] + degp_ref[1, :, 0:1] + 1.0
    dis = lax.rsqrt(deg)
    dis_ref[...] = jnp.broadcast_to(dis, (_NP, 64))
    o_ref[0:_N, :] = dis[0:_N] * y_ref[...]
    o_ref[_N:, :] = jnp.zeros((_NP - _N, o_ref.shape[1]), jnp.float32)


def _bn_relu_scale_body(dis_ref, p_ref, g_ref, be_ref, b_ref, o_ref):
    dis = dis_ref[0:_N, :]
    t = dis * (p_ref[0, 0:_N, :] + p_ref[1, 0:_N, :]) + b_ref[...]
    mu = jnp.mean(t, axis=0, keepdims=True)
    var = jnp.mean((t - mu) ** 2, axis=0, keepdims=True)
    h = (t - mu) * lax.rsqrt(var + 1e-5) * g_ref[...] + be_ref[...]
    o_ref[0:_N, :] = dis * jnp.maximum(h, 0.0)
    o_ref[_N:, :] = jnp.zeros((_NP - _N, o_ref.shape[1]), jnp.float32)


def _layer2_body(dis_ref, p_ref, w2_ref, b2_ref, g2_ref, be2_ref,
                 w3_ref, o_ref):
    dis = dis_ref[0:_N, :]
    u = dis * (p_ref[0, 0:_N, :] + p_ref[1, 0:_N, :])
    t = jnp.dot(u, w2_ref[...], preferred_element_type=jnp.float32) + b2_ref[...]
    mu = jnp.mean(t, axis=0, keepdims=True)
    var = jnp.mean((t - mu) ** 2, axis=0, keepdims=True)
    h = (t - mu) * lax.rsqrt(var + 1e-5) * g2_ref[...] + be2_ref[...]
    h = jnp.maximum(h, 0.0)
    o_ref[0:_N, 0:10] = dis[:, 0:10] * jnp.dot(
        h, w3_ref[...], preferred_element_type=jnp.float32)
    o_ref[0:_N, 10:16] = jnp.zeros((_N, 6), jnp.float32)
    o_ref[_N:, :] = jnp.zeros((_NP - _N, o_ref.shape[1]), jnp.float32)


def _pool_body(dis_ref, p_ref, batch_ref, b3_ref, o_ref):
    t3 = dis_ref[0:_N, 0:16] * (p_ref[0, 0:_N, :] + p_ref[1, 0:_N, :])
    gids = lax.broadcasted_iota(jnp.int32, (64, 1), 0)
    oht = (gids == batch_ref[...]).astype(jnp.float32)
    sums = jnp.dot(oht, t3, preferred_element_type=jnp.float32)
    counts = jnp.dot(oht, jnp.ones((_N, 1), jnp.float32),
                     preferred_element_type=jnp.float32)
    pooled = sums[:, :10] / jnp.maximum(counts, 1.0) + b3_ref[...]
    m = jnp.max(pooled, axis=1, keepdims=True)
    lse = jnp.log(jnp.sum(jnp.exp(pooled - m), axis=1, keepdims=True)) + m
    o_ref[...] = pooled - lse


def _tc(body, out_shape, *args):
    return pl.pallas_call(body, out_shape=out_shape)(*args)


def kernel(x, edge_index, batch, W1, b1, g1, be1, W2, b2, g2, be2, W3, b3):
    f32 = jnp.float32
    ei3 = edge_index.reshape(2 * _NW, _NBAT, _K)
    ones_k = jnp.ones((_K, 16), f32)
    zeros_n16 = jnp.zeros((_NP, 16), f32)
    zeros_n64 = jnp.zeros((_NP, 64), f32)

    degp = _make_deg()(ei3, ones_k, zeros_n16)
    y1 = _tc(_mm_body, jax.ShapeDtypeStruct((_N, 64), f32), x, W1)
    zt1, dis64 = _tc(_scale_body,
                     (jax.ShapeDtypeStruct((_NP, 64), f32),
                      jax.ShapeDtypeStruct((_NP, 64), f32)),
                     degp, y1)

    p1 = _make_agg(64)(ei3, zt1, zeros_n64)
    zt2 = _tc(_bn_relu_scale_body, jax.ShapeDtypeStruct((_NP, 64), f32),
              dis64, p1, g1.reshape(1, -1), be1.reshape(1, -1),
              b1.reshape(1, -1))

    p2 = _make_agg(64)(ei3, zt2, zeros_n64)
    zt3 = _tc(_layer2_body, jax.ShapeDtypeStruct((_NP, 16), f32),
              dis64, p2, W2, b2.reshape(1, -1), g2.reshape(1, -1),
              be2.reshape(1, -1), W3)

    p3 = _make_agg(16)(ei3, zt3, zeros_n16)
    out = _tc(_pool_body, jax.ShapeDtypeStruct((64, 10), f32),
              dis64, p3, batch.reshape(1, _N), b3.reshape(1, -1))
    return out
